# baseline (speedup 1.0000x reference)
import functools

import jax
import jax.numpy as jnp
from jax import lax
from jax.experimental import pallas as pl
from jax.experimental.pallas import tpu as pltpu
from jax.experimental.pallas import tpu_sc as plsc

N = 10000
E = 160000
F = 256
H = 128
NPAD = 10240
EPAD = 163840
NC = 2
NS = 16
CHUNK = 32

ROWS_PT = NPAD // NS
EDGES_PT = EPAD // NS
CHUNKS_PT = EDGES_PT // CHUNK
DEG_EDGES_PT = EPAD // (NC * NS)
DEG_CHUNKS_PT = DEG_EDGES_PT // CHUNK

_mesh = functools.partial(
    plsc.VectorSubcoreMesh, core_axis_name="c", subcore_axis_name="s"
)


@functools.partial(
    pl.kernel,
    out_type=jax.ShapeDtypeStruct((NC, NPAD, 16), jnp.float32),
    mesh=_mesh(),
    scratch_types=[
        pltpu.VMEM((CHUNK,), jnp.int32),
        pltpu.VMEM((CHUNK, 16), jnp.float32),
        pltpu.VMEM_SHARED((NPAD, 16), jnp.float32),
        pltpu.SemaphoreType.DMA,
    ],
)
def _deg_sc(init2, dst_hbm, out, didx, ones_v, dacc, sem):
    c = lax.axis_index("c")
    s = lax.axis_index("s")
    base = s * ROWS_PT
    pltpu.sync_copy(init2.at[c, pl.ds(base, ROWS_PT), :],
                    dacc.at[pl.ds(base, ROWS_PT), :])
    pltpu.sync_copy(init2.at[0, pl.ds(0, CHUNK), :], ones_v)
    plsc.subcore_barrier()

    def chunk(j, carry):
        ebase = (c * NS + s) * DEG_EDGES_PT + j * CHUNK
        pltpu.sync_copy(dst_hbm.at[pl.ds(ebase, CHUNK)], didx)
        pltpu.sync_copy(ones_v, dacc.at[didx], add=True)
        return carry

    lax.fori_loop(0, DEG_CHUNKS_PT, chunk, 0)
    plsc.subcore_barrier()
    pltpu.sync_copy(dacc.at[pl.ds(base, ROWS_PT), :],
                    out.at[c, pl.ds(base, ROWS_PT), :])


_D = 5
_NQ = 8
_QCH = CHUNKS_PT // _NQ


@functools.partial(
    pl.kernel,
    out_type=(
        jax.ShapeDtypeStruct((NPAD, H), jnp.float32),
        jax.ShapeDtypeStruct((NPAD, H), jnp.float32),
    ),
    mesh=_mesh(),
    scratch_types=[
        pltpu.VMEM((_QCH, CHUNK), jnp.int32),
        pltpu.VMEM((_QCH, CHUNK), jnp.int32),
        pltpu.VMEM((_D, CHUNK, H), jnp.float32),
        pltpu.VMEM_SHARED((NPAD, H), jnp.float32),
        pltpu.SemaphoreType.DMA((_D,)),
        pltpu.SemaphoreType.DMA((_D,)),
    ],
)
def _prop_sc(g_lo, g_hi, src2d, dst2d, out_lo, out_hi,
             sidx, didx, rows, accum, gsem, ssem):
    c = lax.axis_index("c")
    s = lax.axis_index("s")

    def half(g, out):
        base = s * ROWS_PT
        pltpu.sync_copy(g.at[pl.ds(base, ROWS_PT), :],
                        accum.at[pl.ds(base, ROWS_PT), :])
        plsc.subcore_barrier()

        def gather(j, b):
            pltpu.async_copy(g.at[sidx.at[j]], rows.at[b], gsem.at[b])

        def gather_wait(j, b):
            pltpu.make_async_copy(g.at[sidx.at[j]], rows.at[b],
                                  gsem.at[b]).wait()

        def scat(j, b):
            pltpu.async_copy(rows.at[b], accum.at[didx.at[j]], ssem.at[b],
                             add=True)

        def scat_wait(j, b):
            pltpu.make_async_copy(rows.at[b], accum.at[didx.at[j]],
                                  ssem.at[b]).wait()

        for q in range(_NQ):
            qbase = s * CHUNKS_PT + q * _QCH
            pltpu.sync_copy(src2d.at[pl.ds(qbase, _QCH), :], sidx)
            pltpu.sync_copy(dst2d.at[pl.ds(qbase, _QCH), :], didx)

            for b in range(_D - 1):
                gather(b, b)

            def outer(t, carry):
                jbase = t * _D
                for b in range(_D):
                    j = jbase + b
                    jn = j + (_D - 1)
                    bn = (b + _D - 1) % _D
                    gather_wait(j, b)
                    scat(j, b)

                    @pl.when(jnp.logical_and(jn >= _D, jn < _QCH))
                    def _():
                        scat_wait(jn, bn)

                    @pl.when(jn < _QCH)
                    def _():
                        gather(jn, bn)
                return carry

            lax.fori_loop(0, _QCH // _D, outer, 0)
            for b in range(_D):
                scat_wait(b, b)
        plsc.subcore_barrier()
        pltpu.sync_copy(accum.at[pl.ds(base, ROWS_PT), :],
                        out.at[pl.ds(base, ROWS_PT), :])

    @pl.when(c == 0)
    def _():
        half(g_lo, out_lo)

    @pl.when(c == 1)
    def _():
        half(g_hi, out_hi)


_BLK = 1280


def _deg_of(degp_ref):
    return degp_ref[0][:, :1] + degp_ref[1][:, :1]


def _scale_body(power, degp_ref, a_lo_ref, a_hi_ref, o_lo_ref, o_hi_ref):
    deg = _deg_of(degp_ref)
    if power == -0.5:
        sc = lax.rsqrt(deg)
    else:
        sc = 1.0 / deg
    o_lo_ref[...] = a_lo_ref[...] * sc
    o_hi_ref[...] = a_hi_ref[...] * sc


def _make_scale(power):
    return pl.pallas_call(
        functools.partial(_scale_body, power),
        grid=(NPAD // _BLK,),
        in_specs=[
            pl.BlockSpec((NC, _BLK, 16), lambda i: (0, i, 0)),
            pl.BlockSpec((_BLK, H), lambda i: (i, 0)),
            pl.BlockSpec((_BLK, H), lambda i: (i, 0)),
        ],
        out_specs=[
            pl.BlockSpec((_BLK, H), lambda i: (i, 0)),
            pl.BlockSpec((_BLK, H), lambda i: (i, 0)),
        ],
        out_shape=[
            jax.ShapeDtypeStruct((NPAD, H), jnp.float32),
            jax.ShapeDtypeStruct((NPAD, H), jnp.float32),
        ],
    )


_scale_rsqrt = _make_scale(-0.5)
_scale_inv = _make_scale(-1.0)


def _mm_pre(degp_ref, a_lo_ref, a_hi_ref, w_ref, b_ref):
    rs = lax.rsqrt(_deg_of(degp_ref))
    h = jnp.dot(a_lo_ref[...] * rs, w_ref[:H, :],
                preferred_element_type=jnp.float32)
    h += jnp.dot(a_hi_ref[...] * rs, w_ref[H:, :],
                 preferred_element_type=jnp.float32)
    return h + b_ref[...], rs


def _mm_mid_body(degp_ref, a_lo_ref, a_hi_ref, w_ref, b_ref,
                 o_lo_ref, o_hi_ref):
    h, rs = _mm_pre(degp_ref, a_lo_ref, a_hi_ref, w_ref, b_ref)
    g = h * rs
    o_lo_ref[...] = g[:, :H]
    o_hi_ref[...] = g[:, H:]


def _mm_out_body(degp_ref, a_lo_ref, a_hi_ref, w_ref, b_ref, o_ref):
    h, _ = _mm_pre(degp_ref, a_lo_ref, a_hi_ref, w_ref, b_ref)
    m = jnp.max(h, axis=1, keepdims=True)
    e = jnp.exp(h - m)
    o_ref[...] = (h - m) - jnp.log(jnp.sum(e, axis=1, keepdims=True))


_mm_in_specs = [
    pl.BlockSpec((NC, _BLK, 16), lambda i: (0, i, 0)),
    pl.BlockSpec((_BLK, H), lambda i: (i, 0)),
    pl.BlockSpec((_BLK, H), lambda i: (i, 0)),
    pl.BlockSpec((F, F), lambda i: (0, 0)),
    pl.BlockSpec((1, F), lambda i: (0, 0)),
]

_mm_mid = pl.pallas_call(
    _mm_mid_body,
    grid=(NPAD // _BLK,),
    in_specs=_mm_in_specs,
    out_specs=[
        pl.BlockSpec((_BLK, H), lambda i: (i, 0)),
        pl.BlockSpec((_BLK, H), lambda i: (i, 0)),
    ],
    out_shape=[
        jax.ShapeDtypeStruct((NPAD, H), jnp.float32),
        jax.ShapeDtypeStruct((NPAD, H), jnp.float32),
    ],
)

_mm_out = pl.pallas_call(
    _mm_out_body,
    grid=(NPAD // _BLK,),
    in_specs=_mm_in_specs,
    out_specs=pl.BlockSpec((_BLK, F), lambda i: (i, 0)),
    out_shape=jax.ShapeDtypeStruct((NPAD, F), jnp.float32),
)


def kernel(x, edge_index, W1, b1, W2, b2):
    src = edge_index[0]
    dst = edge_index[1]
    src_p = jnp.concatenate([src, jnp.zeros((EPAD - E,), jnp.int32)])
    dst_p = jnp.concatenate([dst, jnp.full((EPAD - E,), N, jnp.int32)])
    src2d = src_p.reshape(EPAD // CHUNK, CHUNK)
    dst2d = dst_p.reshape(EPAD // CHUNK, CHUNK)
    x_p = jnp.pad(x, ((0, NPAD - N), (0, 0)))
    x_lo = x_p[:, :H]
    x_hi = x_p[:, H:]
    init2 = jnp.stack([jnp.ones((NPAD, 16), jnp.float32),
                       jnp.zeros((NPAD, 16), jnp.float32)])

    degp = _deg_sc(init2, dst_p)

    g_lo, g_hi = _scale_rsqrt(degp, x_lo, x_hi)
    a_lo, a_hi = _prop_sc(g_lo, g_hi, src2d, dst2d)
    g_lo, g_hi = _scale_inv(degp, a_lo, a_hi)
    a_lo, a_hi = _prop_sc(g_lo, g_hi, src2d, dst2d)
    g_lo, g_hi = _mm_mid(degp, a_lo, a_hi, W1, b1.reshape(1, F))
    a_lo, a_hi = _prop_sc(g_lo, g_hi, src2d, dst2d)
    g_lo, g_hi = _scale_inv(degp, a_lo, a_hi)
    a_lo, a_hi = _prop_sc(g_lo, g_hi, src2d, dst2d)
    out = _mm_out(degp, a_lo, a_hi, W2, b2.reshape(1, F))
    return out[:N]


# speedup vs baseline: 1.0620x; 1.0620x over previous
import functools

import jax
import jax.numpy as jnp
from jax import lax
from jax.experimental import pallas as pl
from jax.experimental.pallas import tpu as pltpu
from jax.experimental.pallas import tpu_sc as plsc

N = 10000
E = 160000
F = 256
H = 128
NPAD = 10240
EPAD = 163840
NC = 2
NS = 16
CHUNK = 64

ROWS_PT = NPAD // NS
EDGES_PT = EPAD // NS
CHUNKS_PT = EDGES_PT // CHUNK
DEG_EDGES_PT = EPAD // (NC * NS)
DEG_CHUNK = 128
DEG_CHUNKS_PT = DEG_EDGES_PT // DEG_CHUNK

_mesh = functools.partial(
    plsc.VectorSubcoreMesh, core_axis_name="c", subcore_axis_name="s"
)


@functools.partial(
    pl.kernel,
    out_type=jax.ShapeDtypeStruct((NC, NPAD, 16), jnp.float32),
    mesh=_mesh(),
    scratch_types=[
        pltpu.VMEM((DEG_CHUNKS_PT, DEG_CHUNK), jnp.int32),
        pltpu.VMEM((DEG_CHUNK, 16), jnp.float32),
        pltpu.VMEM_SHARED((NPAD, 16), jnp.float32),
        pltpu.SemaphoreType.DMA,
    ],
)
def _deg_sc(dst2d, init2, out, didx, ones_v, dacc, sem):
    c = lax.axis_index("c")
    s = lax.axis_index("s")
    base = s * ROWS_PT
    pltpu.sync_copy(init2.at[c, pl.ds(base, ROWS_PT), :],
                    dacc.at[pl.ds(base, ROWS_PT), :])
    pltpu.sync_copy(init2.at[0, pl.ds(0, DEG_CHUNK), :], ones_v)
    cbase = (c * NS + s) * DEG_CHUNKS_PT
    pltpu.sync_copy(dst2d.at[pl.ds(cbase, DEG_CHUNKS_PT), :], didx)
    plsc.subcore_barrier()

    for j in range(DEG_CHUNKS_PT):
        pltpu.async_copy(ones_v, dacc.at[didx.at[j]], sem, add=True)
    for j in range(DEG_CHUNKS_PT):
        pltpu.make_async_copy(ones_v, dacc.at[didx.at[j]], sem).wait()
    plsc.subcore_barrier()
    pltpu.sync_copy(dacc.at[pl.ds(base, ROWS_PT), :],
                    out.at[c, pl.ds(base, ROWS_PT), :])


_D = 4
_NQ = 4
_QCH = CHUNKS_PT // _NQ


@functools.partial(
    pl.kernel,
    out_type=(
        jax.ShapeDtypeStruct((NPAD, H), jnp.float32),
        jax.ShapeDtypeStruct((NPAD, H), jnp.float32),
    ),
    mesh=_mesh(),
    scratch_types=[
        pltpu.VMEM((_QCH, CHUNK), jnp.int32),
        pltpu.VMEM((_QCH, CHUNK), jnp.int32),
        pltpu.VMEM((_D, CHUNK, H), jnp.float32),
        pltpu.VMEM_SHARED((NPAD, H), jnp.float32),
        pltpu.SemaphoreType.DMA((_D,)),
        pltpu.SemaphoreType.DMA((_D,)),
    ],
)
def _prop_sc(g_lo, g_hi, src2d, dst2d, out_lo, out_hi,
             sidx, didx, rows, accum, gsem, ssem):
    c = lax.axis_index("c")
    s = lax.axis_index("s")

    def half(g, out):
        base = s * ROWS_PT
        pltpu.sync_copy(g.at[pl.ds(base, ROWS_PT), :],
                        accum.at[pl.ds(base, ROWS_PT), :])
        plsc.subcore_barrier()

        def gather(j, b):
            pltpu.async_copy(g.at[sidx.at[j]], rows.at[b], gsem.at[b])

        def gather_wait(j, b):
            pltpu.make_async_copy(g.at[sidx.at[j]], rows.at[b],
                                  gsem.at[b]).wait()

        def scat(j, b):
            pltpu.async_copy(rows.at[b], accum.at[didx.at[j]], ssem.at[b],
                             add=True)

        def scat_wait(j, b):
            pltpu.make_async_copy(rows.at[b], accum.at[didx.at[j]],
                                  ssem.at[b]).wait()

        for q in range(_NQ):
            qbase = s * CHUNKS_PT + q * _QCH
            pltpu.sync_copy(src2d.at[pl.ds(qbase, _QCH), :], sidx)
            pltpu.sync_copy(dst2d.at[pl.ds(qbase, _QCH), :], didx)

            for b in range(_D - 1):
                gather(b, b)

            def outer(t, carry):
                jbase = t * _D
                for b in range(_D):
                    j = jbase + b
                    jn = j + (_D - 1)
                    bn = (b + _D - 1) % _D
                    gather_wait(j, b)
                    scat(j, b)

                    @pl.when(jnp.logical_and(jn >= _D, jn < _QCH))
                    def _():
                        scat_wait(jn, bn)

                    @pl.when(jn < _QCH)
                    def _():
                        gather(jn, bn)
                return carry

            lax.fori_loop(0, _QCH // _D, outer, 0)
            for b in range(_D):
                scat_wait(b, b)
        plsc.subcore_barrier()
        pltpu.sync_copy(accum.at[pl.ds(base, ROWS_PT), :],
                        out.at[pl.ds(base, ROWS_PT), :])

    @pl.when(c == 0)
    def _():
        half(g_lo, out_lo)

    @pl.when(c == 1)
    def _():
        half(g_hi, out_hi)


_BLK = 1280


def _deg_of(degp_ref):
    return degp_ref[0][:, :1] + degp_ref[1][:, :1]


def _scale_body(power, degp_ref, a_lo_ref, a_hi_ref, o_lo_ref, o_hi_ref):
    deg = _deg_of(degp_ref)
    if power == -0.5:
        sc = lax.rsqrt(deg)
    else:
        sc = 1.0 / deg
    o_lo_ref[...] = a_lo_ref[...] * sc
    o_hi_ref[...] = a_hi_ref[...] * sc


def _make_scale(power):
    return pl.pallas_call(
        functools.partial(_scale_body, power),
        grid=(NPAD // _BLK,),
        in_specs=[
            pl.BlockSpec((NC, _BLK, 16), lambda i: (0, i, 0)),
            pl.BlockSpec((_BLK, H), lambda i: (i, 0)),
            pl.BlockSpec((_BLK, H), lambda i: (i, 0)),
        ],
        out_specs=[
            pl.BlockSpec((_BLK, H), lambda i: (i, 0)),
            pl.BlockSpec((_BLK, H), lambda i: (i, 0)),
        ],
        out_shape=[
            jax.ShapeDtypeStruct((NPAD, H), jnp.float32),
            jax.ShapeDtypeStruct((NPAD, H), jnp.float32),
        ],
    )


_scale_rsqrt = _make_scale(-0.5)
_scale_inv = _make_scale(-1.0)


def _mm_pre(degp_ref, a_lo_ref, a_hi_ref, w_ref, b_ref):
    rs = lax.rsqrt(_deg_of(degp_ref))
    h = jnp.dot(a_lo_ref[...] * rs, w_ref[:H, :],
                preferred_element_type=jnp.float32)
    h += jnp.dot(a_hi_ref[...] * rs, w_ref[H:, :],
                 preferred_element_type=jnp.float32)
    return h + b_ref[...], rs


def _mm_mid_body(degp_ref, a_lo_ref, a_hi_ref, w_ref, b_ref,
                 o_lo_ref, o_hi_ref):
    h, rs = _mm_pre(degp_ref, a_lo_ref, a_hi_ref, w_ref, b_ref)
    g = h * rs
    o_lo_ref[...] = g[:, :H]
    o_hi_ref[...] = g[:, H:]


def _mm_out_body(degp_ref, a_lo_ref, a_hi_ref, w_ref, b_ref, o_ref):
    h, _ = _mm_pre(degp_ref, a_lo_ref, a_hi_ref, w_ref, b_ref)
    m = jnp.max(h, axis=1, keepdims=True)
    e = jnp.exp(h - m)
    o_ref[...] = (h - m) - jnp.log(jnp.sum(e, axis=1, keepdims=True))


_mm_in_specs = [
    pl.BlockSpec((NC, _BLK, 16), lambda i: (0, i, 0)),
    pl.BlockSpec((_BLK, H), lambda i: (i, 0)),
    pl.BlockSpec((_BLK, H), lambda i: (i, 0)),
    pl.BlockSpec((F, F), lambda i: (0, 0)),
    pl.BlockSpec((1, F), lambda i: (0, 0)),
]

_mm_mid = pl.pallas_call(
    _mm_mid_body,
    grid=(NPAD // _BLK,),
    in_specs=_mm_in_specs,
    out_specs=[
        pl.BlockSpec((_BLK, H), lambda i: (i, 0)),
        pl.BlockSpec((_BLK, H), lambda i: (i, 0)),
    ],
    out_shape=[
        jax.ShapeDtypeStruct((NPAD, H), jnp.float32),
        jax.ShapeDtypeStruct((NPAD, H), jnp.float32),
    ],
)

_mm_out = pl.pallas_call(
    _mm_out_body,
    grid=(NPAD // _BLK,),
    in_specs=_mm_in_specs,
    out_specs=pl.BlockSpec((_BLK, F), lambda i: (i, 0)),
    out_shape=jax.ShapeDtypeStruct((NPAD, F), jnp.float32),
)


def kernel(x, edge_index, W1, b1, W2, b2):
    src = edge_index[0]
    dst = edge_index[1]
    src_p = jnp.concatenate([src, jnp.zeros((EPAD - E,), jnp.int32)])
    dst_p = jnp.concatenate([dst, jnp.full((EPAD - E,), N, jnp.int32)])
    src2d = src_p.reshape(EPAD // CHUNK, CHUNK)
    dst2d = dst_p.reshape(EPAD // CHUNK, CHUNK)
    x_p = jnp.pad(x, ((0, NPAD - N), (0, 0)))
    x_lo = x_p[:, :H]
    x_hi = x_p[:, H:]
    init2 = jnp.stack([jnp.ones((NPAD, 16), jnp.float32),
                       jnp.zeros((NPAD, 16), jnp.float32)])

    degp = _deg_sc(dst_p.reshape(EPAD // DEG_CHUNK, DEG_CHUNK), init2)

    g_lo, g_hi = _scale_rsqrt(degp, x_lo, x_hi)
    a_lo, a_hi = _prop_sc(g_lo, g_hi, src2d, dst2d)
    g_lo, g_hi = _scale_inv(degp, a_lo, a_hi)
    a_lo, a_hi = _prop_sc(g_lo, g_hi, src2d, dst2d)
    g_lo, g_hi = _mm_mid(degp, a_lo, a_hi, W1, b1.reshape(1, F))
    a_lo, a_hi = _prop_sc(g_lo, g_hi, src2d, dst2d)
    g_lo, g_hi = _scale_inv(degp, a_lo, a_hi)
    a_lo, a_hi = _prop_sc(g_lo, g_hi, src2d, dst2d)
    out = _mm_out(degp, a_lo, a_hi, W2, b2.reshape(1, F))
    return out[:N]
